# baseline (speedup 1.0000x reference)
import functools

import jax
import jax.numpy as jnp
from jax import lax
from jax.experimental import pallas as pl
from jax.experimental.pallas import tpu as pltpu
from jax.experimental.pallas import tpu_sc as plsc

_M = 4
_LANES = 16
_NWORKERS = 32
_ROWS = 8
_WIN = _M * _LANES
_NBUF = 2


def _sc_body(x_hbm, o_hbm, in0, in1, out0, out1, si0, si1, so0, so1):
    n, d = x_hbm.shape
    chunk = _ROWS * d
    rows_per_w = n // _NWORKERS
    n_chunks = rows_per_w // _ROWS
    ins = (in0, in1)
    outs = (out0, out1)
    sis = (si0, si1)
    sos = (so0, so1)
    wid = lax.axis_index("s") * 2 + lax.axis_index("c")
    row0 = wid * rows_per_w
    lane4 = lax.iota(jnp.int32, _LANES) * _M
    zero = jnp.zeros((_LANES,), jnp.float32)

    def load(ci, b):
        r = row0 + ci * _ROWS
        pltpu.make_async_copy(x_hbm.at[pl.ds(r, _ROWS)], ins[b], sis[b]).start()

    def store(ci, b):
        r = row0 + ci * _ROWS
        pltpu.make_async_copy(outs[b], o_hbm.at[pl.ds(r, _ROWS)], sos[b]).start()

    def wait_in(b):
        pltpu.make_async_copy(x_hbm.at[pl.ds(row0, _ROWS)], ins[b], sis[b]).wait()

    def wait_out(b):
        pltpu.make_async_copy(outs[b], o_hbm.at[pl.ds(row0, _ROWS)], sos[b]).wait()

    lane = lax.iota(jnp.int32, _LANES)
    perm1 = lane ^ 1
    perm2 = lane ^ 2

    def _xl(v, p):
        return v[p]

    def compute(b):
        src = ins[b]
        dst = outs[b]

        def row_body(r, _):
            @plsc.parallel_loop(0, d, step=_WIN, unroll=4)
            def _(c0):
                for k in range(_WIN // _LANES):
                    c = c0 + k * _LANES
                    v = src[r, pl.ds(c, _LANES)]
                    bb = jnp.abs(v)
                    p = _xl(bb, perm1)
                    pmx = jnp.maximum(bb, p)
                    pmn = jnp.minimum(bb, p)
                    q1 = _xl(pmx, perm2)
                    q2 = _xl(pmn, perm2)
                    second = jnp.maximum(
                        jnp.minimum(pmx, q1), jnp.maximum(pmn, q2)
                    )
                    dst[r, pl.ds(c, _LANES)] = jnp.where(bb >= second, v, zero)

            return 0

        lax.fori_loop(0, _ROWS, row_body, 0)

    for b in range(_NBUF):
        load(b, b)

    def g_body(g, _):
        for b in range(_NBUF):
            ci = g * _NBUF + b
            wait_in(b)

            @pl.when(g > 0)
            def _():
                wait_out(b)

            compute(b)

            @pl.when(ci + _NBUF < n_chunks)
            def _():
                load(ci + _NBUF, b)

            store(ci, b)
        return 0

    lax.fori_loop(0, n_chunks // _NBUF, g_body, 0)
    for b in range(_NBUF):
        wait_out(b)


def kernel(input):
    n, d = input.shape
    assert n % (_NWORKERS * _ROWS * _NBUF) == 0 and d % _WIN == 0
    mesh = plsc.VectorSubcoreMesh(core_axis_name="c", subcore_axis_name="s")
    return pl.kernel(
        _sc_body,
        out_type=jax.ShapeDtypeStruct((n, d), jnp.float32),
        mesh=mesh,
        scratch_types=[pltpu.VMEM((_ROWS, d), jnp.float32)] * 4
        + [pltpu.SemaphoreType.DMA] * 4,
        compiler_params=pltpu.CompilerParams(
            needs_layout_passes=False, use_tc_tiling_on_sc=True
        ),
    )(input)


# speedup vs baseline: 1.4437x; 1.4437x over previous
import functools

import jax
import jax.numpy as jnp
from jax import lax
from jax.experimental import pallas as pl
from jax.experimental.pallas import tpu as pltpu
from jax.experimental.pallas import tpu_sc as plsc

_M = 4
_LANES = 16
_NWORKERS = 32
_ROWS = 8
_WIN = _M * _LANES
_NBUF = 3


def _sc_body(
    x_hbm, o_hbm, in0, in1, in2, out0, out1, out2, si0, si1, si2, so0, so1, so2
):
    n, d = x_hbm.shape
    chunk = _ROWS * d
    rows_per_w = n // _NWORKERS
    n_chunks = rows_per_w // _ROWS
    ins = (in0, in1, in2)
    outs = (out0, out1, out2)
    sis = (si0, si1, si2)
    sos = (so0, so1, so2)
    wid = lax.axis_index("s") * 2 + lax.axis_index("c")
    row0 = wid * rows_per_w
    lane4 = lax.iota(jnp.int32, _LANES) * _M
    zero = jnp.zeros((_LANES,), jnp.float32)

    def load(ci, b):
        r = row0 + ci * _ROWS
        pltpu.make_async_copy(x_hbm.at[pl.ds(r, _ROWS)], ins[b], sis[b]).start()

    def store(ci, b):
        r = row0 + ci * _ROWS
        pltpu.make_async_copy(outs[b], o_hbm.at[pl.ds(r, _ROWS)], sos[b]).start()

    def wait_in(b):
        pltpu.make_async_copy(x_hbm.at[pl.ds(row0, _ROWS)], ins[b], sis[b]).wait()

    def wait_out(b):
        pltpu.make_async_copy(outs[b], o_hbm.at[pl.ds(row0, _ROWS)], sos[b]).wait()

    def compute(b):
        src = ins[b]
        dst = outs[b]

        @plsc.parallel_loop(0, chunk, step=_WIN, unroll=4)
        def _(i):
            r = jnp.full((_LANES,), i // d, jnp.int32)
            i0 = lane4 + i % d
            a0 = plsc.load_gather(src, [r, i0])
            a1 = plsc.load_gather(src, [r, i0 + 1])
            a2 = plsc.load_gather(src, [r, i0 + 2])
            a3 = plsc.load_gather(src, [r, i0 + 3])
            b0 = jnp.abs(a0)
            b1 = jnp.abs(a1)
            b2 = jnp.abs(a2)
            b3 = jnp.abs(a3)
            m1 = jnp.maximum(b0, b1)
            n1 = jnp.minimum(b0, b1)
            m2 = jnp.maximum(b2, b3)
            n2 = jnp.minimum(b2, b3)
            second = jnp.maximum(jnp.minimum(m1, m2), jnp.maximum(n1, n2))
            plsc.store_scatter(dst, [r, i0], jnp.where(b0 >= second, a0, zero))
            plsc.store_scatter(dst, [r, i0 + 1], jnp.where(b1 >= second, a1, zero))
            plsc.store_scatter(dst, [r, i0 + 2], jnp.where(b2 >= second, a2, zero))
            plsc.store_scatter(dst, [r, i0 + 3], jnp.where(b3 >= second, a3, zero))

    for b in range(_NBUF):
        load(b, b)

    def g_body(g, _):
        for b in range(_NBUF):
            ci = g * _NBUF + b
            wait_in(b)

            @pl.when(g > 0)
            def _():
                wait_out(b)

            compute(b)
            store(ci, b)

            @pl.when(ci + _NBUF < n_chunks)
            def _():
                load(ci + _NBUF, b)

        return 0

    lax.fori_loop(0, n_chunks // _NBUF, g_body, 0)
    for ci in range((n_chunks // _NBUF) * _NBUF, n_chunks):
        b = ci % _NBUF
        wait_in(b)
        wait_out(b)
        compute(b)
        store(ci, b)
    for b in range(_NBUF):
        wait_out(b)


def kernel(input):
    n, d = input.shape
    assert n % (_NWORKERS * _ROWS) == 0 and d % _WIN == 0
    mesh = plsc.VectorSubcoreMesh(core_axis_name="c", subcore_axis_name="s")
    return pl.kernel(
        _sc_body,
        out_type=jax.ShapeDtypeStruct((n, d), jnp.float32),
        mesh=mesh,
        scratch_types=[pltpu.VMEM((_ROWS, d), jnp.float32)] * (2 * _NBUF)
        + [pltpu.SemaphoreType.DMA] * (2 * _NBUF),
        compiler_params=pltpu.CompilerParams(
            needs_layout_passes=False, use_tc_tiling_on_sc=True
        ),
    )(input)


# speedup vs baseline: 1.4457x; 1.0014x over previous
import jax
import jax.numpy as jnp
from jax import lax
from jax.experimental import pallas as pl
from jax.experimental.pallas import tpu as pltpu
from jax.experimental.pallas import tpu_sc as plsc

_M = 4
_LANES = 16
_NWORKERS = 32
_ROWS = 8
_WIN = _M * _LANES
_NBUF = 3


def _sc_body(
    x_hbm, o_hbm, in0, in1, in2, out0, out1, out2, si0, si1, si2, so0, so1, so2
):
    n, d = x_hbm.shape
    chunk = _ROWS * d
    rows_per_w = n // _NWORKERS
    n_chunks = rows_per_w // _ROWS
    ins = (in0, in1, in2)
    outs = (out0, out1, out2)
    sis = (si0, si1, si2)
    sos = (so0, so1, so2)
    wid = lax.axis_index("s") * 2 + lax.axis_index("c")
    row0 = wid * rows_per_w
    lane4 = lax.iota(jnp.int32, _LANES) * _M
    zero = jnp.zeros((_LANES,), jnp.float32)

    def load(ci, b):
        r = row0 + ci * _ROWS
        pltpu.make_async_copy(x_hbm.at[pl.ds(r, _ROWS)], ins[b], sis[b]).start()

    def store(ci, b):
        r = row0 + ci * _ROWS
        pltpu.make_async_copy(outs[b], o_hbm.at[pl.ds(r, _ROWS)], sos[b]).start()

    def wait_in(b):
        pltpu.make_async_copy(x_hbm.at[pl.ds(row0, _ROWS)], ins[b], sis[b]).wait()

    def wait_out(b):
        pltpu.make_async_copy(outs[b], o_hbm.at[pl.ds(row0, _ROWS)], sos[b]).wait()

    def compute(b):
        src = ins[b]
        dst = outs[b]

        @plsc.parallel_loop(0, chunk, step=_WIN, unroll=4)
        def _(i):
            r = jnp.full((_LANES,), i // d, jnp.int32)
            i0 = lane4 + i % d
            a0 = plsc.load_gather(src, [r, i0])
            a1 = plsc.load_gather(src, [r, i0 + 1])
            a2 = plsc.load_gather(src, [r, i0 + 2])
            a3 = plsc.load_gather(src, [r, i0 + 3])
            b0 = jnp.abs(a0)
            b1 = jnp.abs(a1)
            b2 = jnp.abs(a2)
            b3 = jnp.abs(a3)
            m1 = jnp.maximum(b0, b1)
            n1 = jnp.minimum(b0, b1)
            m2 = jnp.maximum(b2, b3)
            n2 = jnp.minimum(b2, b3)
            second = jnp.maximum(jnp.minimum(m1, m2), jnp.maximum(n1, n2))
            plsc.store_scatter(dst, [r, i0], jnp.where(b0 >= second, a0, zero))
            plsc.store_scatter(dst, [r, i0 + 1], jnp.where(b1 >= second, a1, zero))
            plsc.store_scatter(dst, [r, i0 + 2], jnp.where(b2 >= second, a2, zero))
            plsc.store_scatter(dst, [r, i0 + 3], jnp.where(b3 >= second, a3, zero))

    for b in range(_NBUF):
        load(b, b)

    def g_body(g, _):
        for b in range(_NBUF):
            ci = g * _NBUF + b
            wait_in(b)

            @pl.when(g > 0)
            def _():
                wait_out(b)

            compute(b)
            store(ci, b)

            @pl.when(ci + _NBUF < n_chunks)
            def _():
                load(ci + _NBUF, b)

        return 0

    lax.fori_loop(0, n_chunks // _NBUF, g_body, 0)
    for ci in range((n_chunks // _NBUF) * _NBUF, n_chunks):
        b = ci % _NBUF
        wait_in(b)
        wait_out(b)
        compute(b)
        store(ci, b)
    for b in range(_NBUF):
        wait_out(b)


def kernel(input):
    n, d = input.shape
    assert n % (_NWORKERS * _ROWS) == 0 and d % _WIN == 0
    assert n >= _NWORKERS * _ROWS * _NBUF
    mesh = plsc.VectorSubcoreMesh(core_axis_name="c", subcore_axis_name="s")
    return pl.kernel(
        _sc_body,
        out_type=jax.ShapeDtypeStruct((n, d), jnp.float32),
        mesh=mesh,
        scratch_types=[pltpu.VMEM((_ROWS, d), jnp.float32)] * (2 * _NBUF)
        + [pltpu.SemaphoreType.DMA] * (2 * _NBUF),
        compiler_params=pltpu.CompilerParams(
            needs_layout_passes=False, use_tc_tiling_on_sc=True
        ),
    )(input)
